# baseline (speedup 1.0000x reference)
import functools

import jax
import jax.numpy as jnp
from jax import lax
from jax.experimental import pallas as pl
from jax.experimental.pallas import tpu as pltpu
from jax.experimental.pallas import tpu_sc as plsc

B = 1024
L = 200
F = 16
V = 100000

LANES = 16
LPAD = 208
NVEC = LPAD // LANES
PAD_ID = V
TABLE_W = 100096
TROWS = 256
OUT_W = L * F
STG_W = LPAD * F


def _lut_tc_kernel(w1_ref, b1_ref, w2_ref, b2_ref, t_ref):
    c = lax.broadcasted_iota(jnp.int32, (F, TROWS), 1).astype(jnp.float32)
    h = jnp.maximum(w1_ref[...] * c + b1_ref[...], 0.0)
    t = lax.dot_general(w2_ref[...], h, (((1,), (0,)), ((), ())),
                        preferred_element_type=jnp.float32)
    t_ref[...] = t + b2_ref[...]


def _build_lut(W1, b1, W2, b2):
    return pl.pallas_call(
        _lut_tc_kernel,
        out_shape=jax.ShapeDtypeStruct((F, TROWS), jnp.float32),
    )(W1.reshape(F, 1), b1.reshape(F, 1), W2, b2.reshape(F, 1))


def _sc_kernel(rows_per, src_hbm, dst_hbm, lut_hbm, out_s_hbm, out_d_hbm,
               table, ids_s, ids_d, c_ss, c_sd, c_dd, c_ds, lut, stg_s, stg_d):
    wid = lax.axis_index("s") * 2 + lax.axis_index("c")
    pltpu.sync_copy(lut_hbm, lut)

    zeros16 = jnp.zeros((LANES,), jnp.int32)
    ones16 = jnp.ones((LANES,), jnp.int32)
    iota16 = lax.iota(jnp.int32, LANES)
    ii16 = iota16 * F

    def zinit(i, carry):
        for u in range(8):
            table[pl.ds((i * 8 + u) * LANES, LANES)] = zeros16
        return carry
    lax.fori_loop(0, TABLE_W // (8 * LANES), zinit, 0)

    def row_body(r, carry):
        row = wid * rows_per + r
        pltpu.sync_copy(src_hbm.at[row], ids_s)
        pltpu.sync_copy(dst_hbm.at[row], ids_d)

        def bld_s(v, c):
            plsc.addupdate_scatter(table, [ids_s[pl.ds(v * LANES, LANES)]], ones16)
            return c
        lax.fori_loop(0, NVEC, bld_s, 0)

        def gat_s(v, c):
            sv = ids_s[pl.ds(v * LANES, LANES)]
            dv = ids_d[pl.ds(v * LANES, LANES)]
            cs = plsc.load_gather(table, [sv])
            cd = plsc.load_gather(table, [dv])
            c_ss[pl.ds(v * LANES, LANES)] = jnp.where(sv == 0, 0, cs)
            c_ds[pl.ds(v * LANES, LANES)] = jnp.where(dv == 0, 0, cd)
            return c
        lax.fori_loop(0, NVEC, gat_s, 0)

        def clr_s(v, c):
            plsc.store_scatter(table, [ids_s[pl.ds(v * LANES, LANES)]], zeros16)
            return c
        lax.fori_loop(0, NVEC, clr_s, 0)

        def bld_d(v, c):
            plsc.addupdate_scatter(table, [ids_d[pl.ds(v * LANES, LANES)]], ones16)
            return c
        lax.fori_loop(0, NVEC, bld_d, 0)

        def gat_d(v, c):
            sv = ids_s[pl.ds(v * LANES, LANES)]
            dv = ids_d[pl.ds(v * LANES, LANES)]
            cd = plsc.load_gather(table, [dv])
            cs = plsc.load_gather(table, [sv])
            c_dd[pl.ds(v * LANES, LANES)] = jnp.where(dv == 0, 0, cd)
            c_sd[pl.ds(v * LANES, LANES)] = jnp.where(sv == 0, 0, cs)
            return c
        lax.fori_loop(0, NVEC, gat_d, 0)

        def clr_d(v, c):
            plsc.store_scatter(table, [ids_d[pl.ds(v * LANES, LANES)]], zeros16)
            return c
        lax.fori_loop(0, NVEC, clr_d, 0)

        def mlp(v, c):
            lim = jnp.full((LANES,), TROWS - 1, jnp.int32)
            c1 = jnp.minimum(c_ss[pl.ds(v * LANES, LANES)], lim)
            c2 = jnp.minimum(c_sd[pl.ds(v * LANES, LANES)], lim)
            c3 = jnp.minimum(c_dd[pl.ds(v * LANES, LANES)], lim)
            c4 = jnp.minimum(c_ds[pl.ds(v * LANES, LANES)], lim)
            base = v * (LANES * F)
            for f in range(F):
                fs = jnp.full((LANES,), f, jnp.int32)
                vs = plsc.load_gather(lut, [fs, c1]) + plsc.load_gather(lut, [fs, c2])
                vd = plsc.load_gather(lut, [fs, c3]) + plsc.load_gather(lut, [fs, c4])
                oidx = ii16 + (base + f)
                plsc.store_scatter(stg_s, [oidx], vs)
                plsc.store_scatter(stg_d, [oidx], vd)
            return c
        lax.fori_loop(0, NVEC, mlp, 0)

        pltpu.sync_copy(stg_s.at[pl.ds(0, OUT_W)], out_s_hbm.at[row])
        pltpu.sync_copy(stg_d.at[pl.ds(0, OUT_W)], out_d_hbm.at[row])
        return carry
    lax.fori_loop(0, rows_per, row_body, 0)


@jax.jit
def kernel(src_ids, dst_ids, W1, b1, W2, b2):
    lut = _build_lut(W1, b1, W2, b2)

    pad = jnp.full((B, LPAD - L), PAD_ID, jnp.int32)
    src_p = jnp.concatenate([src_ids, pad], axis=1)
    dst_p = jnp.concatenate([dst_ids, pad], axis=1)

    info = plsc.get_sparse_core_info()
    nw = info.num_cores * info.num_subcores
    rows_per = B // nw

    mesh = plsc.VectorSubcoreMesh(core_axis_name="c", subcore_axis_name="s")
    sck = functools.partial(
        pl.kernel,
        mesh=mesh,
        compiler_params=pltpu.CompilerParams(needs_layout_passes=False),
        out_type=[jax.ShapeDtypeStruct((B, OUT_W), jnp.float32),
                  jax.ShapeDtypeStruct((B, OUT_W), jnp.float32)],
        scratch_types=[
            pltpu.VMEM((TABLE_W,), jnp.int32),
            pltpu.VMEM((LPAD,), jnp.int32),
            pltpu.VMEM((LPAD,), jnp.int32),
            pltpu.VMEM((LPAD,), jnp.int32),
            pltpu.VMEM((LPAD,), jnp.int32),
            pltpu.VMEM((LPAD,), jnp.int32),
            pltpu.VMEM((LPAD,), jnp.int32),
            pltpu.VMEM((F, TROWS), jnp.float32),
            pltpu.VMEM((STG_W,), jnp.float32),
            pltpu.VMEM((STG_W,), jnp.float32),
        ],
    )(functools.partial(_sc_kernel, rows_per))
    out_s, out_d = sck(src_p, dst_p, lut)
    return out_s.reshape(B, L, F), out_d.reshape(B, L, F)


# speedup vs baseline: 1.0427x; 1.0427x over previous
import functools

import jax
import jax.numpy as jnp
from jax import lax
from jax.experimental import pallas as pl
from jax.experimental.pallas import tpu as pltpu
from jax.experimental.pallas import tpu_sc as plsc

B = 1024
L = 200
F = 16
V = 100000

LANES = 16
LPAD = 208
NVEC = LPAD // LANES
PAD_ID = V
TABLE_W = 100096
TROWS = 256
OUT_W = L * F
STG_W = LPAD * F


def _lut_tc_kernel(w1_ref, b1_ref, w2_ref, b2_ref, t_ref):
    c = lax.broadcasted_iota(jnp.int32, (F, TROWS), 1).astype(jnp.float32)
    h = jnp.maximum(w1_ref[...] * c + b1_ref[...], 0.0)
    t = lax.dot_general(w2_ref[...], h, (((1,), (0,)), ((), ())),
                        preferred_element_type=jnp.float32)
    t_ref[...] = t + b2_ref[...]


def _build_lut(W1, b1, W2, b2):
    return pl.pallas_call(
        _lut_tc_kernel,
        out_shape=jax.ShapeDtypeStruct((F, TROWS), jnp.float32),
    )(W1.reshape(F, 1), b1.reshape(F, 1), W2, b2.reshape(F, 1))


def _sc_kernel(rows_per, src_hbm, dst_hbm, lut_hbm, out_s_hbm, out_d_hbm,
               table, ids_s, ids_d, c_ss, c_sd, c_dd, c_ds, lut, stg_s, stg_d):
    wid = lax.axis_index("s") * 2 + lax.axis_index("c")
    pltpu.sync_copy(lut_hbm, lut)

    zeros16 = jnp.zeros((LANES,), jnp.int32)
    ones16 = jnp.ones((LANES,), jnp.int32)
    iota16 = lax.iota(jnp.int32, LANES)
    ii16 = iota16 * F

    def zinit(i, carry):
        for u in range(8):
            table[pl.ds((i * 8 + u) * LANES, LANES)] = zeros16
        return carry
    lax.fori_loop(0, TABLE_W // (8 * LANES), zinit, 0)

    lim = jnp.full((LANES,), TROWS - 1, jnp.int32)

    def row_body(r, carry):
        row = wid * rows_per + r
        pltpu.sync_copy(src_hbm.at[row], ids_s)
        pltpu.sync_copy(dst_hbm.at[row], ids_d)

        sv = [ids_s[pl.ds(v * LANES, LANES)] for v in range(NVEC)]
        dv = [ids_d[pl.ds(v * LANES, LANES)] for v in range(NVEC)]

        for v in range(NVEC):
            plsc.addupdate_scatter(table, [sv[v]], ones16)
        for v in range(NVEC):
            cs = plsc.load_gather(table, [sv[v]])
            cd = plsc.load_gather(table, [dv[v]])
            c_ss[pl.ds(v * LANES, LANES)] = jnp.minimum(jnp.where(sv[v] == 0, 0, cs), lim)
            c_ds[pl.ds(v * LANES, LANES)] = jnp.minimum(jnp.where(dv[v] == 0, 0, cd), lim)
        for v in range(NVEC):
            plsc.store_scatter(table, [sv[v]], zeros16)

        for v in range(NVEC):
            plsc.addupdate_scatter(table, [dv[v]], ones16)
        for v in range(NVEC):
            cd = plsc.load_gather(table, [dv[v]])
            cs = plsc.load_gather(table, [sv[v]])
            c_dd[pl.ds(v * LANES, LANES)] = jnp.minimum(jnp.where(dv[v] == 0, 0, cd), lim)
            c_sd[pl.ds(v * LANES, LANES)] = jnp.minimum(jnp.where(sv[v] == 0, 0, cs), lim)
        for v in range(NVEC):
            plsc.store_scatter(table, [dv[v]], zeros16)

        def mlp(v, c):
            c1 = c_ss[pl.ds(v * LANES, LANES)]
            c2 = c_sd[pl.ds(v * LANES, LANES)]
            c3 = c_dd[pl.ds(v * LANES, LANES)]
            c4 = c_ds[pl.ds(v * LANES, LANES)]
            base = v * (LANES * F)
            for f in range(F):
                fs = jnp.full((LANES,), f, jnp.int32)
                vs = plsc.load_gather(lut, [fs, c1]) + plsc.load_gather(lut, [fs, c2])
                vd = plsc.load_gather(lut, [fs, c3]) + plsc.load_gather(lut, [fs, c4])
                oidx = ii16 + (base + f)
                plsc.store_scatter(stg_s, [oidx], vs)
                plsc.store_scatter(stg_d, [oidx], vd)
            return c
        lax.fori_loop(0, NVEC, mlp, 0)

        pltpu.sync_copy(stg_s.at[pl.ds(0, OUT_W)], out_s_hbm.at[row])
        pltpu.sync_copy(stg_d.at[pl.ds(0, OUT_W)], out_d_hbm.at[row])
        return carry
    lax.fori_loop(0, rows_per, row_body, 0)


@jax.jit
def kernel(src_ids, dst_ids, W1, b1, W2, b2):
    lut = _build_lut(W1, b1, W2, b2)

    pad = jnp.full((B, LPAD - L), PAD_ID, jnp.int32)
    src_p = jnp.concatenate([src_ids, pad], axis=1)
    dst_p = jnp.concatenate([dst_ids, pad], axis=1)

    info = plsc.get_sparse_core_info()
    nw = info.num_cores * info.num_subcores
    rows_per = B // nw

    mesh = plsc.VectorSubcoreMesh(core_axis_name="c", subcore_axis_name="s")
    sck = functools.partial(
        pl.kernel,
        mesh=mesh,
        compiler_params=pltpu.CompilerParams(needs_layout_passes=False),
        out_type=[jax.ShapeDtypeStruct((B, OUT_W), jnp.float32),
                  jax.ShapeDtypeStruct((B, OUT_W), jnp.float32)],
        scratch_types=[
            pltpu.VMEM((TABLE_W,), jnp.int32),
            pltpu.VMEM((LPAD,), jnp.int32),
            pltpu.VMEM((LPAD,), jnp.int32),
            pltpu.VMEM((LPAD,), jnp.int32),
            pltpu.VMEM((LPAD,), jnp.int32),
            pltpu.VMEM((LPAD,), jnp.int32),
            pltpu.VMEM((LPAD,), jnp.int32),
            pltpu.VMEM((F, TROWS), jnp.float32),
            pltpu.VMEM((STG_W,), jnp.float32),
            pltpu.VMEM((STG_W,), jnp.float32),
        ],
    )(functools.partial(_sc_kernel, rows_per))
    out_s, out_d = sck(src_p, dst_p, lut)
    return out_s.reshape(B, L, F), out_d.reshape(B, L, F)


# speedup vs baseline: 1.3300x; 1.2756x over previous
import functools

import jax
import jax.numpy as jnp
from jax import lax
from jax.experimental import pallas as pl
from jax.experimental.pallas import tpu as pltpu
from jax.experimental.pallas import tpu_sc as plsc

B = 1024
L = 200
F = 16
V = 100000

LANES = 16
LPAD = 208
NVEC = LPAD // LANES
PAD_ID = V
TABLE_W = 100096
TROWS = 256
OUT_W = L * F
STG_W = LPAD * F


def _lut_tc_kernel(w1_ref, b1_ref, w2_ref, b2_ref, t_ref):
    c = lax.broadcasted_iota(jnp.int32, (F, TROWS), 1).astype(jnp.float32)
    h = jnp.maximum(w1_ref[...] * c + b1_ref[...], 0.0)
    t = lax.dot_general(w2_ref[...], h, (((1,), (0,)), ((), ())),
                        preferred_element_type=jnp.float32)
    t_ref[...] = t + b2_ref[...]


def _build_lut(W1, b1, W2, b2):
    return pl.pallas_call(
        _lut_tc_kernel,
        out_shape=jax.ShapeDtypeStruct((F, TROWS), jnp.float32),
    )(W1.reshape(F, 1), b1.reshape(F, 1), W2, b2.reshape(F, 1))


def _sc_kernel(rows_per, src_hbm, dst_hbm, lut_hbm, out_s_hbm, out_d_hbm,
               table, ids_sA, ids_dA, ids_sB, ids_dB,
               c_ss, c_sd, c_dd, c_ds, lut,
               stg_sA, stg_dA, stg_sB, stg_dB,
               sem_isA, sem_idA, sem_isB, sem_idB,
               sem_osA, sem_odA, sem_osB, sem_odB):
    wid = lax.axis_index("s") * 2 + lax.axis_index("c")
    base_row = wid * rows_per
    n_pairs = rows_per // 2
    pltpu.sync_copy(lut_hbm, lut)

    zeros16 = jnp.zeros((LANES,), jnp.int32)
    ones16 = jnp.ones((LANES,), jnp.int32)
    iota16 = lax.iota(jnp.int32, LANES)
    ii16 = iota16 * F
    lim = jnp.full((LANES,), TROWS - 1, jnp.int32)

    def zinit(i, carry):
        for u in range(8):
            table[pl.ds((i * 8 + u) * LANES, LANES)] = zeros16
        return carry
    lax.fori_loop(0, TABLE_W // (8 * LANES), zinit, 0)

    pltpu.async_copy(src_hbm.at[base_row], ids_sA, sem_isA)
    pltpu.async_copy(dst_hbm.at[base_row], ids_dA, sem_idA)
    pltpu.async_copy(src_hbm.at[base_row + 1], ids_sB, sem_isB)
    pltpu.async_copy(dst_hbm.at[base_row + 1], ids_dB, sem_idB)

    def process(k, row, ids_s, ids_d, sem_is, sem_id, stg_s, stg_d,
                sem_os, sem_od):
        pltpu.make_async_copy(src_hbm.at[row], ids_s, sem_is).wait()
        pltpu.make_async_copy(dst_hbm.at[row], ids_d, sem_id).wait()

        sv = [ids_s[pl.ds(v * LANES, LANES)] for v in range(NVEC)]
        dv = [ids_d[pl.ds(v * LANES, LANES)] for v in range(NVEC)]

        @pl.when(k + 1 < n_pairs)
        def _():
            pltpu.async_copy(src_hbm.at[row + 2], ids_s, sem_is)
            pltpu.async_copy(dst_hbm.at[row + 2], ids_d, sem_id)

        @pl.when(k > 0)
        def _():
            pltpu.make_async_copy(stg_s.at[pl.ds(0, OUT_W)],
                                  out_s_hbm.at[row - 2], sem_os).wait()
            pltpu.make_async_copy(stg_d.at[pl.ds(0, OUT_W)],
                                  out_d_hbm.at[row - 2], sem_od).wait()

        for v in range(NVEC):
            plsc.addupdate_scatter(table, [sv[v]], ones16)
        for v in range(NVEC):
            cs = plsc.load_gather(table, [sv[v]])
            cd = plsc.load_gather(table, [dv[v]])
            c_ss[pl.ds(v * LANES, LANES)] = jnp.minimum(jnp.where(sv[v] == 0, 0, cs), lim)
            c_ds[pl.ds(v * LANES, LANES)] = jnp.minimum(jnp.where(dv[v] == 0, 0, cd), lim)
        for v in range(NVEC):
            plsc.store_scatter(table, [sv[v]], zeros16)

        for v in range(NVEC):
            plsc.addupdate_scatter(table, [dv[v]], ones16)
        for v in range(NVEC):
            cd = plsc.load_gather(table, [dv[v]])
            cs = plsc.load_gather(table, [sv[v]])
            c_dd[pl.ds(v * LANES, LANES)] = jnp.minimum(jnp.where(dv[v] == 0, 0, cd), lim)
            c_sd[pl.ds(v * LANES, LANES)] = jnp.minimum(jnp.where(sv[v] == 0, 0, cs), lim)
        for v in range(NVEC):
            plsc.store_scatter(table, [dv[v]], zeros16)

        def mlp(v, c):
            c1 = c_ss[pl.ds(v * LANES, LANES)]
            c2 = c_sd[pl.ds(v * LANES, LANES)]
            c3 = c_dd[pl.ds(v * LANES, LANES)]
            c4 = c_ds[pl.ds(v * LANES, LANES)]
            base = v * (LANES * F)
            for f in range(F):
                fs = jnp.full((LANES,), f, jnp.int32)
                vs = plsc.load_gather(lut, [fs, c1]) + plsc.load_gather(lut, [fs, c2])
                vd = plsc.load_gather(lut, [fs, c3]) + plsc.load_gather(lut, [fs, c4])
                oidx = ii16 + (base + f)
                plsc.store_scatter(stg_s, [oidx], vs)
                plsc.store_scatter(stg_d, [oidx], vd)
            return c
        lax.fori_loop(0, NVEC, mlp, 0)

        pltpu.async_copy(stg_s.at[pl.ds(0, OUT_W)], out_s_hbm.at[row], sem_os)
        pltpu.async_copy(stg_d.at[pl.ds(0, OUT_W)], out_d_hbm.at[row], sem_od)

    def pair_body(k, carry):
        row_a = base_row + 2 * k
        process(k, row_a, ids_sA, ids_dA, sem_isA, sem_idA,
                stg_sA, stg_dA, sem_osA, sem_odA)
        process(k, row_a + 1, ids_sB, ids_dB, sem_isB, sem_idB,
                stg_sB, stg_dB, sem_osB, sem_odB)
        return carry
    lax.fori_loop(0, n_pairs, pair_body, 0)

    last_a = base_row + rows_per - 2
    pltpu.make_async_copy(stg_sA.at[pl.ds(0, OUT_W)], out_s_hbm.at[last_a], sem_osA).wait()
    pltpu.make_async_copy(stg_dA.at[pl.ds(0, OUT_W)], out_d_hbm.at[last_a], sem_odA).wait()
    pltpu.make_async_copy(stg_sB.at[pl.ds(0, OUT_W)], out_s_hbm.at[last_a + 1], sem_osB).wait()
    pltpu.make_async_copy(stg_dB.at[pl.ds(0, OUT_W)], out_d_hbm.at[last_a + 1], sem_odB).wait()


@jax.jit
def kernel(src_ids, dst_ids, W1, b1, W2, b2):
    lut = _build_lut(W1, b1, W2, b2)

    pad = jnp.full((B, LPAD - L), PAD_ID, jnp.int32)
    src_p = jnp.concatenate([src_ids, pad], axis=1)
    dst_p = jnp.concatenate([dst_ids, pad], axis=1)

    info = plsc.get_sparse_core_info()
    nw = info.num_cores * info.num_subcores
    rows_per = B // nw

    mesh = plsc.VectorSubcoreMesh(core_axis_name="c", subcore_axis_name="s")
    sck = functools.partial(
        pl.kernel,
        mesh=mesh,
        compiler_params=pltpu.CompilerParams(needs_layout_passes=False),
        out_type=[jax.ShapeDtypeStruct((B, OUT_W), jnp.float32),
                  jax.ShapeDtypeStruct((B, OUT_W), jnp.float32)],
        scratch_types=(
            [pltpu.VMEM((TABLE_W,), jnp.int32)]
            + [pltpu.VMEM((LPAD,), jnp.int32) for _ in range(4)]
            + [pltpu.VMEM((LPAD,), jnp.int32) for _ in range(4)]
            + [pltpu.VMEM((F, TROWS), jnp.float32)]
            + [pltpu.VMEM((STG_W,), jnp.float32) for _ in range(4)]
            + [pltpu.SemaphoreType.DMA for _ in range(8)]
        ),
    )(functools.partial(_sc_kernel, rows_per))
    out_s, out_d = sck(src_p, dst_p, lut)
    return out_s.reshape(B, L, F), out_d.reshape(B, L, F)


# speedup vs baseline: 2.0890x; 1.5706x over previous
import functools

import jax
import jax.numpy as jnp
from jax import lax
from jax.experimental import pallas as pl
from jax.experimental.pallas import tpu as pltpu
from jax.experimental.pallas import tpu_sc as plsc

B = 1024
L = 200
F = 16
V = 100000

LANES = 16
LPAD = 208
NVEC = LPAD // LANES
PAD_ID = V
TABLE_W = 100096
CW = 4 * LPAD
BBLK = 128


def _sc_kernel(rows_per, src_hbm, dst_hbm, cnt_hbm,
               table, ids_sA, ids_dA, ids_sB, ids_dB, cntA, cntB,
               sem_isA, sem_idA, sem_isB, sem_idB, sem_cA, sem_cB):
    wid = lax.axis_index("s") * 2 + lax.axis_index("c")
    base_row = wid * rows_per
    n_pairs = rows_per // 2

    zeros16 = jnp.zeros((LANES,), jnp.int32)
    ones16 = jnp.ones((LANES,), jnp.int32)

    def zinit(i, carry):
        for u in range(8):
            table[pl.ds((i * 8 + u) * LANES, LANES)] = zeros16
        return carry
    lax.fori_loop(0, TABLE_W // (8 * LANES), zinit, 0)

    pltpu.async_copy(src_hbm.at[base_row], ids_sA, sem_isA)
    pltpu.async_copy(dst_hbm.at[base_row], ids_dA, sem_idA)
    pltpu.async_copy(src_hbm.at[base_row + 1], ids_sB, sem_isB)
    pltpu.async_copy(dst_hbm.at[base_row + 1], ids_dB, sem_idB)

    def process(k, row, ids_s, ids_d, sem_is, sem_id, cnt, sem_c):
        pltpu.make_async_copy(src_hbm.at[row], ids_s, sem_is).wait()
        pltpu.make_async_copy(dst_hbm.at[row], ids_d, sem_id).wait()

        sv = [ids_s[pl.ds(v * LANES, LANES)] for v in range(NVEC)]
        dv = [ids_d[pl.ds(v * LANES, LANES)] for v in range(NVEC)]

        @pl.when(k + 1 < n_pairs)
        def _():
            pltpu.async_copy(src_hbm.at[row + 2], ids_s, sem_is)
            pltpu.async_copy(dst_hbm.at[row + 2], ids_d, sem_id)

        @pl.when(k > 0)
        def _():
            pltpu.make_async_copy(cnt, cnt_hbm.at[row - 2], sem_c).wait()

        for v in range(NVEC):
            plsc.addupdate_scatter(table, [sv[v]], ones16)
        for v in range(NVEC):
            cs = plsc.load_gather(table, [sv[v]])
            cd = plsc.load_gather(table, [dv[v]])
            cnt[pl.ds(v * LANES, LANES)] = jnp.where(sv[v] == 0, 0, cs)
            cnt[pl.ds(3 * LPAD + v * LANES, LANES)] = jnp.where(dv[v] == 0, 0, cd)
        for v in range(NVEC):
            plsc.store_scatter(table, [sv[v]], zeros16)

        for v in range(NVEC):
            plsc.addupdate_scatter(table, [dv[v]], ones16)
        for v in range(NVEC):
            cd = plsc.load_gather(table, [dv[v]])
            cs = plsc.load_gather(table, [sv[v]])
            cnt[pl.ds(2 * LPAD + v * LANES, LANES)] = jnp.where(dv[v] == 0, 0, cd)
            cnt[pl.ds(LPAD + v * LANES, LANES)] = jnp.where(sv[v] == 0, 0, cs)
        for v in range(NVEC):
            plsc.store_scatter(table, [dv[v]], zeros16)

        pltpu.async_copy(cnt, cnt_hbm.at[row], sem_c)

    def pair_body(k, carry):
        row_a = base_row + 2 * k
        process(k, row_a, ids_sA, ids_dA, sem_isA, sem_idA, cntA, sem_cA)
        process(k, row_a + 1, ids_sB, ids_dB, sem_isB, sem_idB, cntB, sem_cB)
        return carry
    lax.fori_loop(0, n_pairs, pair_body, 0)

    last_a = base_row + rows_per - 2
    pltpu.make_async_copy(cntA, cnt_hbm.at[last_a], sem_cA).wait()
    pltpu.make_async_copy(cntB, cnt_hbm.at[last_a + 1], sem_cB).wait()


def _mlp_tc_kernel(ct_ref, w1_ref, b1_ref, w2_ref, b2_ref, os_ref, od_ref):
    ct = ct_ref[...].astype(jnp.float32)
    c1 = ct[0:LPAD]
    c2 = ct[LPAD:2 * LPAD]
    c3 = ct[2 * LPAD:3 * LPAD]
    c4 = ct[3 * LPAD:4 * LPAD]
    for ca, cb, oref in ((c1, c2, os_ref), (c3, c4, od_ref)):
        hs = []
        for g in range(F):
            w1g = w1_ref[g, 0]
            b1g = b1_ref[g, 0]
            hs.append(jnp.maximum(w1g * ca + b1g, 0.0)
                      + jnp.maximum(w1g * cb + b1g, 0.0))
        for f in range(F):
            acc = hs[0] * w2_ref[f, 0]
            for g in range(1, F):
                acc = acc + hs[g] * w2_ref[f, g]
            acc = acc + 2.0 * b2_ref[f, 0]
            oref[:, f, :] = acc[0:L]


def _mlp_tc(ct, W1, b1, W2, b2):
    nblk = B // BBLK
    return pl.pallas_call(
        _mlp_tc_kernel,
        grid=(nblk,),
        in_specs=[
            pl.BlockSpec((CW, BBLK), lambda i: (0, i)),
            pl.BlockSpec((F, 1), lambda i: (0, 0)),
            pl.BlockSpec((F, 1), lambda i: (0, 0)),
            pl.BlockSpec((F, F), lambda i: (0, 0)),
            pl.BlockSpec((F, 1), lambda i: (0, 0)),
        ],
        out_specs=[
            pl.BlockSpec((L, F, BBLK), lambda i: (0, 0, i)),
            pl.BlockSpec((L, F, BBLK), lambda i: (0, 0, i)),
        ],
        out_shape=[jax.ShapeDtypeStruct((L, F, B), jnp.float32),
                   jax.ShapeDtypeStruct((L, F, B), jnp.float32)],
    )(ct, W1.reshape(F, 1), b1.reshape(F, 1), W2, b2.reshape(F, 1))


@jax.jit
def kernel(src_ids, dst_ids, W1, b1, W2, b2):
    pad = jnp.full((B, LPAD - L), PAD_ID, jnp.int32)
    src_p = jnp.concatenate([src_ids, pad], axis=1)
    dst_p = jnp.concatenate([dst_ids, pad], axis=1)

    info = plsc.get_sparse_core_info()
    nw = info.num_cores * info.num_subcores
    rows_per = B // nw

    mesh = plsc.VectorSubcoreMesh(core_axis_name="c", subcore_axis_name="s")
    sck = functools.partial(
        pl.kernel,
        mesh=mesh,
        compiler_params=pltpu.CompilerParams(needs_layout_passes=False),
        out_type=jax.ShapeDtypeStruct((B, CW), jnp.int32),
        scratch_types=(
            [pltpu.VMEM((TABLE_W,), jnp.int32)]
            + [pltpu.VMEM((LPAD,), jnp.int32) for _ in range(4)]
            + [pltpu.VMEM((CW,), jnp.int32) for _ in range(2)]
            + [pltpu.SemaphoreType.DMA for _ in range(6)]
        ),
    )(functools.partial(_sc_kernel, rows_per))
    counts = sck(src_p, dst_p)

    ct = jnp.transpose(counts, (1, 0))
    os_lfb, od_lfb = _mlp_tc(ct, W1, b1, W2, b2)
    return (jnp.transpose(os_lfb, (2, 0, 1)),
            jnp.transpose(od_lfb, (2, 0, 1)))


# speedup vs baseline: 3.7148x; 1.7783x over previous
import functools

import jax
import jax.numpy as jnp
from jax import lax
from jax.experimental import pallas as pl
from jax.experimental.pallas import tpu as pltpu
from jax.experimental.pallas import tpu_sc as plsc

B = 1024
L = 200
F = 16
V = 100000

LANES = 16
LPAD = 208
NVEC = LPAD // LANES
PAD_ID = V
TABLE_W = 100096
CW = 4 * LPAD
BBLK = 256


def _sc_kernel(rows_per, src_hbm, dst_hbm, cnt_hbm,
               table, ids_sA, ids_dA, ids_sB, ids_dB, cntA, cntB,
               sem_isA, sem_idA, sem_isB, sem_idB, sem_cA, sem_cB):
    wid = lax.axis_index("s") * 2 + lax.axis_index("c")
    base_row = wid * rows_per
    n_pairs = rows_per // 2

    zeros16 = jnp.zeros((LANES,), jnp.int32)
    ones16 = jnp.ones((LANES,), jnp.int32)

    def zinit(i, carry):
        for u in range(8):
            table[pl.ds((i * 8 + u) * LANES, LANES)] = zeros16
        return carry
    lax.fori_loop(0, TABLE_W // (8 * LANES), zinit, 0)

    pltpu.async_copy(src_hbm.at[base_row], ids_sA, sem_isA)
    pltpu.async_copy(dst_hbm.at[base_row], ids_dA, sem_idA)
    pltpu.async_copy(src_hbm.at[base_row + 1], ids_sB, sem_isB)
    pltpu.async_copy(dst_hbm.at[base_row + 1], ids_dB, sem_idB)

    def process(k, row, ids_s, ids_d, sem_is, sem_id, cnt, sem_c):
        pltpu.make_async_copy(src_hbm.at[row], ids_s, sem_is).wait()
        pltpu.make_async_copy(dst_hbm.at[row], ids_d, sem_id).wait()

        sv = [ids_s[pl.ds(v * LANES, LANES)] for v in range(NVEC)]
        dv = [ids_d[pl.ds(v * LANES, LANES)] for v in range(NVEC)]

        @pl.when(k + 1 < n_pairs)
        def _():
            pltpu.async_copy(src_hbm.at[row + 2], ids_s, sem_is)
            pltpu.async_copy(dst_hbm.at[row + 2], ids_d, sem_id)

        @pl.when(k > 0)
        def _():
            pltpu.make_async_copy(cnt, cnt_hbm.at[row - 2], sem_c).wait()

        for v in range(NVEC):
            plsc.addupdate_scatter(table, [sv[v]], ones16)
        for v in range(NVEC):
            cs = plsc.load_gather(table, [sv[v]])
            cd = plsc.load_gather(table, [dv[v]])
            cnt[pl.ds(v * LANES, LANES)] = jnp.where(sv[v] == 0, 0, cs)
            cnt[pl.ds(3 * LPAD + v * LANES, LANES)] = jnp.where(dv[v] == 0, 0, cd)
        for v in range(NVEC):
            plsc.store_scatter(table, [sv[v]], zeros16)

        for v in range(NVEC):
            plsc.addupdate_scatter(table, [dv[v]], ones16)
        for v in range(NVEC):
            cd = plsc.load_gather(table, [dv[v]])
            cs = plsc.load_gather(table, [sv[v]])
            cnt[pl.ds(2 * LPAD + v * LANES, LANES)] = jnp.where(dv[v] == 0, 0, cd)
            cnt[pl.ds(LPAD + v * LANES, LANES)] = jnp.where(sv[v] == 0, 0, cs)
        for v in range(NVEC):
            plsc.store_scatter(table, [dv[v]], zeros16)

        pltpu.async_copy(cnt, cnt_hbm.at[row], sem_c)

    def pair_body(k, carry):
        row_a = base_row + 2 * k
        process(k, row_a, ids_sA, ids_dA, sem_isA, sem_idA, cntA, sem_cA)
        process(k, row_a + 1, ids_sB, ids_dB, sem_isB, sem_idB, cntB, sem_cB)
        return carry
    lax.fori_loop(0, n_pairs, pair_body, 0)

    last_a = base_row + rows_per - 2
    pltpu.make_async_copy(cntA, cnt_hbm.at[last_a], sem_cA).wait()
    pltpu.make_async_copy(cntB, cnt_hbm.at[last_a + 1], sem_cB).wait()


def _mlp_tc_kernel(ct_ref, w1_ref, b1_ref, w2_ref, b2_ref, os_ref, od_ref):
    ctf = ct_ref[...].astype(jnp.float32)
    w1 = w1_ref[...]
    b1 = b1_ref[...]
    w2 = w2_ref[...]
    b22 = 2.0 * b2_ref[...]
    for qa, qb, oref in ((0, 1, os_ref), (2, 3, od_ref)):
        for l in range(L):
            ca = ctf[qa, l:l + 1, :]
            cb = ctf[qb, l:l + 1, :]
            h = (jnp.maximum(w1 * ca + b1, 0.0)
                 + jnp.maximum(w1 * cb + b1, 0.0))
            y = jax.lax.dot_general(w2, h, (((1,), (0,)), ((), ())),
                                    preferred_element_type=jnp.float32)
            oref[l] = y + b22


def _mlp_tc(ct3, W1, b1, W2, b2):
    nblk = B // BBLK
    return pl.pallas_call(
        _mlp_tc_kernel,
        grid=(nblk,),
        in_specs=[
            pl.BlockSpec((4, LPAD, BBLK), lambda i: (0, 0, i)),
            pl.BlockSpec((F, 1), lambda i: (0, 0)),
            pl.BlockSpec((F, 1), lambda i: (0, 0)),
            pl.BlockSpec((F, F), lambda i: (0, 0)),
            pl.BlockSpec((F, 1), lambda i: (0, 0)),
        ],
        out_specs=[
            pl.BlockSpec((L, F, BBLK), lambda i: (0, 0, i)),
            pl.BlockSpec((L, F, BBLK), lambda i: (0, 0, i)),
        ],
        out_shape=[jax.ShapeDtypeStruct((L, F, B), jnp.float32),
                   jax.ShapeDtypeStruct((L, F, B), jnp.float32)],
    )(ct3, W1.reshape(F, 1), b1.reshape(F, 1), W2, b2.reshape(F, 1))


@jax.jit
def kernel(src_ids, dst_ids, W1, b1, W2, b2):
    pad = jnp.full((B, LPAD - L), PAD_ID, jnp.int32)
    src_p = jnp.concatenate([src_ids, pad], axis=1)
    dst_p = jnp.concatenate([dst_ids, pad], axis=1)

    info = plsc.get_sparse_core_info()
    nw = info.num_cores * info.num_subcores
    rows_per = B // nw

    mesh = plsc.VectorSubcoreMesh(core_axis_name="c", subcore_axis_name="s")
    sck = functools.partial(
        pl.kernel,
        mesh=mesh,
        compiler_params=pltpu.CompilerParams(needs_layout_passes=False),
        out_type=jax.ShapeDtypeStruct((B, CW), jnp.int32),
        scratch_types=(
            [pltpu.VMEM((TABLE_W,), jnp.int32)]
            + [pltpu.VMEM((LPAD,), jnp.int32) for _ in range(4)]
            + [pltpu.VMEM((CW,), jnp.int32) for _ in range(2)]
            + [pltpu.SemaphoreType.DMA for _ in range(6)]
        ),
    )(functools.partial(_sc_kernel, rows_per))
    counts = sck(src_p, dst_p)

    ct3 = jnp.transpose(counts.reshape(B, 4, LPAD), (1, 2, 0))
    os_lfb, od_lfb = _mlp_tc(ct3, W1, b1, W2, b2)
    return (jnp.transpose(os_lfb, (2, 0, 1)),
            jnp.transpose(od_lfb, (2, 0, 1)))
